# baseline (speedup 1.0000x reference)
import functools

import jax
import jax.numpy as jnp
from jax import lax
from jax.experimental import pallas as pl
from jax.experimental.pallas import tpu as pltpu
from jax.experimental.pallas import tpu_sc as plsc

B = 4
N_DUAL = 81920
N_VERTEX = 40962
D = 64
NU = 3

NC = 2
NS = 16
NW = NC * NS

ZCOLS = B * D
OROWS = B * NU * D
OLEN = N_VERTEX
CPW = ZCOLS // NW

CHUNK = 10240
NCHUNK = 4
TAIL = OLEN - NCHUNK * CHUNK
IPAD = 40968
NV8 = CHUNK // 128

_mesh = plsc.VectorSubcoreMesh(core_axis_name="c", subcore_axis_name="s")


@functools.partial(
    pl.kernel,
    out_type=(
        jax.ShapeDtypeStruct((OROWS, OLEN), jnp.float32),
        jax.ShapeDtypeStruct((OROWS, 8), jnp.float32),
    ),
    mesh=_mesh,
    scratch_types=[
        pltpu.VMEM((N_DUAL,), jnp.float32),
        pltpu.VMEM((2, CHUNK), jnp.int32),
        pltpu.VMEM((2, CHUNK), jnp.float32),
        pltpu.VMEM((16,), jnp.int32),
        pltpu.VMEM((16,), jnp.float32),
    ]
    + [pltpu.SemaphoreType.DMA] * 4,
    compiler_params=pltpu.CompilerParams(use_tc_tiling_on_sc=False,
                                         needs_layout_passes=False),
)
def _col_gather(zc_hbm, idx_hbm, out_hbm, stash_hbm,
                zcol_v, idx_v, out_v, tidx_v, tval_v, *sems):
    isem = sems[0:2]
    osem = sems[2:4]
    c = lax.axis_index("c")
    s = lax.axis_index("s")
    wid = s * NC + c
    col0 = wid * CPW

    def start_idx(j, k, sl):
        pltpu.async_copy(idx_hbm.at[j, pl.ds(k * CHUNK, CHUNK)],
                         idx_v.at[sl], isem[sl])

    def wait_idx(sl):
        pltpu.make_async_copy(idx_hbm.at[0, pl.ds(0, CHUNK)],
                              idx_v.at[sl], isem[sl]).wait()

    def start_out(r, k, sl):
        pltpu.async_copy(out_v.at[sl],
                         out_hbm.at[r, pl.ds(k * CHUNK, CHUNK)], osem[sl])

    def wait_out(sl):
        pltpu.make_async_copy(out_v.at[sl],
                              out_hbm.at[0, pl.ds(0, CHUNK)], osem[sl]).wait()

    def gather_chunk(sl):
        def vec8(t, carry):
            for u in range(8):
                o = t * 128 + u * 16
                iv = idx_v[sl, pl.ds(o, 16)]
                out_v[sl, pl.ds(o, 16)] = plsc.load_gather(zcol_v, [iv])
            return carry

        lax.fori_loop(0, NV8, vec8, 0)

    zeros16 = jnp.zeros((16,), jnp.int32)

    def body(p, carry):
        ci = p // NU
        j = lax.rem(p, NU)
        col = col0 + ci
        b = col // D
        d = lax.rem(col, D)
        r = b * (NU * D) + j * D + d

        @pl.when(j == 0)
        def _():
            pltpu.sync_copy(zc_hbm.at[col], zcol_v)

        start_idx(j, 0, 0)
        for k in range(NCHUNK):
            sl = k & 1
            if k + 1 < NCHUNK:
                start_idx(j, k + 1, sl ^ 1)
            wait_idx(sl)
            if k >= 2:
                wait_out(sl)
            gather_chunk(sl)
            start_out(r, k, sl)

        tidx_v[pl.ds(0, 16)] = zeros16
        pltpu.sync_copy(idx_hbm.at[j, pl.ds(NCHUNK * CHUNK, 8)],
                        tidx_v.at[pl.ds(0, 8)])
        tval_v[pl.ds(0, 16)] = plsc.load_gather(zcol_v, [tidx_v[pl.ds(0, 16)]])
        pltpu.sync_copy(tval_v.at[pl.ds(0, 8)], stash_hbm.at[r])

        wait_out(0)
        wait_out(1)
        return carry

    lax.fori_loop(0, CPW * NU, body, 0)


def kernel(z_prime, x_ancil, index):
    del x_ancil
    zc = jnp.transpose(z_prime, (0, 2, 1)).reshape(ZCOLS, N_DUAL)
    idx_t = jnp.transpose(index.astype(jnp.int32), (1, 0))
    idx_p = jnp.pad(idx_t, ((0, 0), (0, IPAD - OLEN)))
    out_t, stash = _col_gather(zc, idx_p)
    out_t = out_t.at[:, NCHUNK * CHUNK:].set(stash[:, :TAIL])
    return jnp.transpose(out_t.reshape(B, NU * D, N_VERTEX), (0, 2, 1))


# speedup vs baseline: 2.9518x; 2.9518x over previous
import functools

import jax
import jax.numpy as jnp
from jax import lax
from jax.experimental import pallas as pl
from jax.experimental.pallas import tpu as pltpu
from jax.experimental.pallas import tpu_sc as plsc

B = 4
N_DUAL = 81920
N_VERTEX = 40962
D = 64
NU = 3

NC = 2
NS = 16
NW = NC * NS

VB = N_DUAL // 128
IB = 321
IPAD = IB * 128
CBLK = NU * D // 8

CHB = (80, 80, 80, 81)
CHOFF = (0, 80, 160, 240)
CBUF = 81

_mesh = plsc.VectorSubcoreMesh(core_axis_name="c", subcore_axis_name="s")


@functools.partial(
    pl.kernel,
    out_type=jax.ShapeDtypeStruct((B, CBLK, IB, 8, 128), jnp.float32),
    mesh=_mesh,
    scratch_types=[
        pltpu.VMEM((VB, 128), jnp.float32),
        pltpu.VMEM((2, CBUF, 128), jnp.int32),
        pltpu.VMEM((2, CBUF, 128), jnp.float32),
    ]
    + [pltpu.SemaphoreType.DMA] * 4,
    compiler_params=pltpu.CompilerParams(use_tc_tiling_on_sc=False,
                                         needs_layout_passes=False),
)
def _col_gather(zv_hbm, idx_hbm, out_hbm, zcol_v, idx_v, out_v, *sems):
    isem = sems[0:2]
    osem = sems[2:4]
    c = lax.axis_index("c")
    s = lax.axis_index("s")
    wid = s * NC + c
    b = wid // 8

    def start_idx(j, k, sl):
        pltpu.async_copy(idx_hbm.at[j, pl.ds(CHOFF[k], CHB[k])],
                         idx_v.at[sl, pl.ds(0, CHB[k])], isem[sl])

    def wait_idx(k, sl):
        pltpu.make_async_copy(idx_hbm.at[0, pl.ds(0, CHB[k])],
                              idx_v.at[sl, pl.ds(0, CHB[k])], isem[sl]).wait()

    def start_out(cb, ci, k, sl):
        pltpu.async_copy(out_v.at[sl, pl.ds(0, CHB[k])],
                         out_hbm.at[b, cb, pl.ds(CHOFF[k], CHB[k]), ci],
                         osem[sl])

    def wait_out(k, sl):
        pltpu.make_async_copy(out_v.at[sl, pl.ds(0, CHB[k])],
                              out_hbm.at[0, 0, pl.ds(0, CHB[k]), 0],
                              osem[sl]).wait()

    def gather_chunk(k, sl):
        def vec8(t, carry):
            for u in range(8):
                iv = idx_v[sl, t, pl.ds(u * 16, 16)]
                hi = lax.shift_right_logical(iv, 7)
                lo = lax.bitwise_and(iv, 127)
                out_v[sl, t, pl.ds(u * 16, 16)] = plsc.load_gather(
                    zcol_v, [hi, lo])
            return carry

        lax.fori_loop(0, CHB[k], vec8, 0)

    def body(p, carry):
        ci = p // NU
        j = lax.rem(p, NU)
        cb = j * 8 + lax.rem(wid, 8)

        @pl.when(j == 0)
        def _():
            pltpu.sync_copy(zv_hbm.at[wid, pl.ds(0, VB), ci], zcol_v)

        start_idx(j, 0, 0)
        for k in range(4):
            sl = k & 1
            if k < 3:
                start_idx(j, k + 1, sl ^ 1)
            wait_idx(k, sl)
            if k >= 2:
                wait_out(k - 2, sl)
            gather_chunk(k, sl)
            start_out(cb, ci, k, sl)
        wait_out(2, 0)
        wait_out(3, 1)
        return carry

    lax.fori_loop(0, 8 * NU, body, 0)


def kernel(z_prime, x_ancil, index):
    del x_ancil
    zv = z_prime.reshape(B, VB, 128, 8, 8).transpose(0, 3, 1, 4, 2)
    zv = zv.reshape(NW, VB, 8, 128)
    idx_t = jnp.transpose(index.astype(jnp.int32), (1, 0))
    idx_p = jnp.pad(idx_t, ((0, 0), (0, IPAD - N_VERTEX))).reshape(NU, IB, 128)
    out5 = _col_gather(zv, idx_p)
    out = out5.transpose(0, 1, 3, 2, 4).reshape(B, NU * D, IPAD)
    return out[:, :, :N_VERTEX].transpose(0, 2, 1)


# speedup vs baseline: 4.3484x; 1.4731x over previous
import functools

import jax
import jax.numpy as jnp
from jax import lax
from jax.experimental import pallas as pl
from jax.experimental.pallas import tpu as pltpu
from jax.experimental.pallas import tpu_sc as plsc

B = 4
N_DUAL = 81920
N_VERTEX = 40962
D = 64
NU = 3

NC = 2
NS = 16
NW = NC * NS

VB = N_DUAL // 128
IB = 321
IPAD = IB * 128
CBLK = NU * D // 8

CHB = (80, 80, 80, 81)
CHOFF = (0, 80, 160, 240)
CBUF = 81

_mesh = plsc.VectorSubcoreMesh(core_axis_name="c", subcore_axis_name="s")


@functools.partial(
    pl.kernel,
    out_type=jax.ShapeDtypeStruct((B, CBLK, IB, 8, 128), jnp.float32),
    mesh=_mesh,
    scratch_types=[
        pltpu.VMEM((VB, 128), jnp.float32),
        pltpu.VMEM((2, CBUF, 128), jnp.int32),
        pltpu.VMEM((2, CBUF, 128), jnp.float32),
    ]
    + [pltpu.SemaphoreType.DMA] * 4,
    compiler_params=pltpu.CompilerParams(use_tc_tiling_on_sc=False,
                                         needs_layout_passes=False),
)
def _col_gather(zv_hbm, idx_hbm, out_hbm, zcol_v, idx_v, out_v, *sems):
    isem = sems[0:2]
    osem = sems[2:4]
    c = lax.axis_index("c")
    s = lax.axis_index("s")
    wid = s * NC + c
    b = wid // 8

    def start_idx(j, k, sl):
        pltpu.async_copy(idx_hbm.at[j, pl.ds(CHOFF[k], CHB[k])],
                         idx_v.at[sl, pl.ds(0, CHB[k])], isem[sl])

    def wait_idx(k, sl):
        pltpu.make_async_copy(idx_hbm.at[0, pl.ds(0, CHB[k])],
                              idx_v.at[sl, pl.ds(0, CHB[k])], isem[sl]).wait()

    def start_out(cb, ci, k, sl):
        pltpu.async_copy(out_v.at[sl, pl.ds(0, CHB[k])],
                         out_hbm.at[b, cb, pl.ds(CHOFF[k], CHB[k]), ci],
                         osem[sl])

    def wait_out(k, sl):
        pltpu.make_async_copy(out_v.at[sl, pl.ds(0, CHB[k])],
                              out_hbm.at[0, 0, pl.ds(0, CHB[k]), 0],
                              osem[sl]).wait()

    def gather_chunk(k, sl):
        @plsc.parallel_loop(0, CHB[k], unroll=2)
        def _vec8(t):
            for u in range(8):
                iv = idx_v[sl, t, pl.ds(u * 16, 16)]
                hi = lax.shift_right_logical(iv, 7)
                lo = lax.bitwise_and(iv, 127)
                out_v[sl, t, pl.ds(u * 16, 16)] = plsc.load_gather(
                    zcol_v, [hi, lo])

    def body(p, carry):
        ci = p // NU
        j = lax.rem(p, NU)
        cb = j * 8 + lax.rem(wid, 8)

        @pl.when(j == 0)
        def _():
            pltpu.sync_copy(zv_hbm.at[wid, pl.ds(0, VB), ci], zcol_v)

        start_idx(j, 0, 0)
        for k in range(4):
            sl = k & 1
            if k < 3:
                start_idx(j, k + 1, sl ^ 1)
            wait_idx(k, sl)
            if k >= 2:
                wait_out(k - 2, sl)
            gather_chunk(k, sl)
            start_out(cb, ci, k, sl)
        wait_out(2, 0)
        wait_out(3, 1)
        return carry

    lax.fori_loop(0, 8 * NU, body, 0)


def kernel(z_prime, x_ancil, index):
    del x_ancil
    zv = z_prime.reshape(B, VB, 128, 8, 8).transpose(0, 3, 1, 4, 2)
    zv = zv.reshape(NW, VB, 8, 128)
    idx_t = jnp.transpose(index.astype(jnp.int32), (1, 0))
    idx_p = jnp.pad(idx_t, ((0, 0), (0, IPAD - N_VERTEX))).reshape(NU, IB, 128)
    out5 = _col_gather(zv, idx_p)
    out = out5.transpose(0, 1, 3, 2, 4).reshape(B, NU * D, IPAD)
    return out[:, :, :N_VERTEX].transpose(0, 2, 1)


# speedup vs baseline: 4.3583x; 1.0023x over previous
import functools

import jax
import jax.numpy as jnp
from jax import lax
from jax.experimental import pallas as pl
from jax.experimental.pallas import tpu as pltpu
from jax.experimental.pallas import tpu_sc as plsc

B = 4
N_DUAL = 81920
N_VERTEX = 40962
D = 64
NU = 3

NC = 2
NS = 16
NW = NC * NS

VB = N_DUAL // 128
IB = 321
IPAD = IB * 128
CBLK = NU * D // 8

CHB = (80, 80, 80, 81)
CHOFF = (0, 80, 160, 240)
CBUF = 81

_mesh = plsc.VectorSubcoreMesh(core_axis_name="c", subcore_axis_name="s")


@functools.partial(
    pl.kernel,
    out_type=jax.ShapeDtypeStruct((B, CBLK, IB, 8, 128), jnp.float32),
    mesh=_mesh,
    scratch_types=[
        pltpu.VMEM((VB, 128), jnp.float32),
        pltpu.VMEM((2, CBUF, 128), jnp.int32),
        pltpu.VMEM((2, CBUF, 128), jnp.float32),
    ]
    + [pltpu.SemaphoreType.DMA] * 4,
    compiler_params=pltpu.CompilerParams(use_tc_tiling_on_sc=False,
                                         needs_layout_passes=False),
)
def _col_gather(zv_hbm, idx_hbm, out_hbm, zcol_v, idx_v, out_v, *sems):
    isem = sems[0:2]
    osem = sems[2:4]
    c = lax.axis_index("c")
    s = lax.axis_index("s")
    wid = s * NC + c
    b = wid // 8

    def start_idx(j, k, sl):
        pltpu.async_copy(idx_hbm.at[j, pl.ds(CHOFF[k], CHB[k])],
                         idx_v.at[sl, pl.ds(0, CHB[k])], isem[sl])

    def wait_idx(k, sl):
        pltpu.make_async_copy(idx_hbm.at[0, pl.ds(0, CHB[k])],
                              idx_v.at[sl, pl.ds(0, CHB[k])], isem[sl]).wait()

    def start_out(cb, ci, k, sl):
        pltpu.async_copy(out_v.at[sl, pl.ds(0, CHB[k])],
                         out_hbm.at[b, cb, pl.ds(CHOFF[k], CHB[k]), ci],
                         osem[sl])

    def wait_out(k, sl):
        pltpu.make_async_copy(out_v.at[sl, pl.ds(0, CHB[k])],
                              out_hbm.at[0, 0, pl.ds(0, CHB[k]), 0],
                              osem[sl]).wait()

    def gather_chunk(k, sl):
        @plsc.parallel_loop(0, CHB[k], unroll=4)
        def _vec8(t):
            for u in range(8):
                iv = idx_v[sl, t, pl.ds(u * 16, 16)]
                hi = lax.shift_right_logical(iv, 7)
                lo = lax.bitwise_and(iv, 127)
                out_v[sl, t, pl.ds(u * 16, 16)] = plsc.load_gather(
                    zcol_v, [hi, lo])

    def body(p, carry):
        ci = p // NU
        j = lax.rem(p, NU)
        cb = j * 8 + lax.rem(wid, 8)

        @pl.when(j == 0)
        def _():
            pltpu.sync_copy(zv_hbm.at[wid, pl.ds(0, VB), ci], zcol_v)

        start_idx(j, 0, 0)
        for k in range(4):
            sl = k & 1
            if k < 3:
                start_idx(j, k + 1, sl ^ 1)
            wait_idx(k, sl)
            if k >= 2:
                wait_out(k - 2, sl)
            gather_chunk(k, sl)
            start_out(cb, ci, k, sl)
        wait_out(2, 0)
        wait_out(3, 1)
        return carry

    lax.fori_loop(0, 8 * NU, body, 0)


def kernel(z_prime, x_ancil, index):
    del x_ancil
    zv = z_prime.reshape(B, VB, 128, 8, 8).transpose(0, 3, 1, 4, 2)
    zv = zv.reshape(NW, VB, 8, 128)
    idx_t = jnp.transpose(index.astype(jnp.int32), (1, 0))
    idx_p = jnp.pad(idx_t, ((0, 0), (0, IPAD - N_VERTEX))).reshape(NU, IB, 128)
    out5 = _col_gather(zv, idx_p)
    out = out5.transpose(0, 1, 3, 2, 4).reshape(B, NU * D, IPAD)
    return out[:, :, :N_VERTEX].transpose(0, 2, 1)


# speedup vs baseline: 4.3786x; 1.0047x over previous
import functools

import jax
import jax.numpy as jnp
from jax import lax
from jax.experimental import pallas as pl
from jax.experimental.pallas import tpu as pltpu
from jax.experimental.pallas import tpu_sc as plsc

B = 4
N_DUAL = 81920
N_VERTEX = 40962
D = 64
NU = 3

NC = 2
NS = 16
NW = NC * NS

VB = N_DUAL // 128
IB = 321
IPAD = IB * 128
CBLK = NU * D // 8

CHB = (80, 80, 80, 81)
CHOFF = (0, 80, 160, 240)
CBUF = 81

_mesh = plsc.VectorSubcoreMesh(core_axis_name="c", subcore_axis_name="s")


@functools.partial(
    pl.kernel,
    out_type=jax.ShapeDtypeStruct((B, CBLK, IB, 8, 128), jnp.float32),
    mesh=_mesh,
    scratch_types=[
        pltpu.VMEM((VB, 128), jnp.float32),
        pltpu.VMEM((2, CBUF, 128), jnp.int32),
        pltpu.VMEM((2, CBUF, 128), jnp.float32),
    ]
    + [pltpu.SemaphoreType.DMA] * 5,
    compiler_params=pltpu.CompilerParams(use_tc_tiling_on_sc=False,
                                         needs_layout_passes=False),
)
def _col_gather(zv_hbm, idx_hbm, out_hbm, zcol_v, idx_v, out_v, *sems):
    isem = sems[0:2]
    osem = sems[2:4]
    zsem = sems[4]
    c = lax.axis_index("c")
    s = lax.axis_index("s")
    wid = s * NC + c
    b = wid // 8

    def start_idx(j, k, sl):
        pltpu.async_copy(idx_hbm.at[j, pl.ds(CHOFF[k], CHB[k])],
                         idx_v.at[sl, pl.ds(0, CHB[k])], isem[sl])

    def wait_idx(k, sl):
        pltpu.make_async_copy(idx_hbm.at[0, pl.ds(0, CHB[k])],
                              idx_v.at[sl, pl.ds(0, CHB[k])], isem[sl]).wait()

    def start_out(cb, ci, k, sl):
        pltpu.async_copy(out_v.at[sl, pl.ds(0, CHB[k])],
                         out_hbm.at[b, cb, pl.ds(CHOFF[k], CHB[k]), ci],
                         osem[sl])

    def wait_out(k, sl):
        pltpu.make_async_copy(out_v.at[sl, pl.ds(0, CHB[k])],
                              out_hbm.at[0, 0, pl.ds(0, CHB[k]), 0],
                              osem[sl]).wait()

    def gather_chunk(k, sl):
        @plsc.parallel_loop(0, CHB[k], unroll=4)
        def _vec8(t):
            for u in range(8):
                iv = idx_v[sl, t, pl.ds(u * 16, 16)]
                hi = lax.shift_right_logical(iv, 7)
                lo = lax.bitwise_and(iv, 127)
                out_v[sl, t, pl.ds(u * 16, 16)] = plsc.load_gather(
                    zcol_v, [hi, lo])

    def wait_z():
        pltpu.make_async_copy(zv_hbm.at[0, pl.ds(0, VB), 0], zcol_v,
                              zsem).wait()

    pltpu.async_copy(zv_hbm.at[wid, pl.ds(0, VB), 0], zcol_v, zsem)

    def body(p, carry):
        ci = p // NU
        j = lax.rem(p, NU)
        cb = j * 8 + lax.rem(wid, 8)

        @pl.when(j == 0)
        def _():
            wait_z()

        start_idx(j, 0, 0)
        for k in range(4):
            sl = k & 1
            if k < 3:
                start_idx(j, k + 1, sl ^ 1)
            wait_idx(k, sl)
            if k >= 2:
                wait_out(k - 2, sl)
            gather_chunk(k, sl)
            start_out(cb, ci, k, sl)

        @pl.when((j == NU - 1) & (ci < 7))
        def _():
            pltpu.async_copy(zv_hbm.at[wid, pl.ds(0, VB), ci + 1], zcol_v,
                             zsem)

        wait_out(2, 0)
        wait_out(3, 1)
        return carry

    lax.fori_loop(0, 8 * NU, body, 0)


def kernel(z_prime, x_ancil, index):
    del x_ancil
    zv = z_prime.reshape(B, VB, 128, 8, 8).transpose(0, 3, 1, 4, 2)
    zv = zv.reshape(NW, VB, 8, 128)
    idx_t = jnp.transpose(index.astype(jnp.int32), (1, 0))
    idx_p = jnp.pad(idx_t, ((0, 0), (0, IPAD - N_VERTEX))).reshape(NU, IB, 128)
    out5 = _col_gather(zv, idx_p)
    out = out5.transpose(0, 1, 3, 2, 4).reshape(B, NU * D, IPAD)
    return out[:, :, :N_VERTEX].transpose(0, 2, 1)


# speedup vs baseline: 4.6581x; 1.0638x over previous
import functools

import jax
import jax.numpy as jnp
from jax import lax
from jax.experimental import pallas as pl
from jax.experimental.pallas import tpu as pltpu
from jax.experimental.pallas import tpu_sc as plsc

B = 4
N_DUAL = 81920
N_VERTEX = 40962
D = 64
NU = 3

NC = 2
NS = 16
NW = NC * NS

VB = N_DUAL // 128
IB = 321
IPAD = IB * 128
CBLK = NU * D // 8

CHB = (80, 80, 80, 81)
CHOFF = (0, 80, 160, 240)
CBUF = 81

_mesh = plsc.VectorSubcoreMesh(core_axis_name="c", subcore_axis_name="s")


@functools.partial(
    pl.kernel,
    out_type=jax.ShapeDtypeStruct((B, CBLK, IB, 8, 128), jnp.float32),
    mesh=_mesh,
    scratch_types=[
        pltpu.VMEM((VB, 128), jnp.float32),
        pltpu.VMEM((2, CBUF, 128), jnp.int32),
        pltpu.VMEM((2, CBUF, 128), jnp.float32),
    ]
    + [pltpu.SemaphoreType.DMA] * 5,
    compiler_params=pltpu.CompilerParams(use_tc_tiling_on_sc=False,
                                         needs_layout_passes=False),
)
def _col_gather(zv_hbm, idx_hbm, out_hbm, zcol_v, idx_v, out_v, *sems):
    isem = sems[0:2]
    osem = sems[2:4]
    zsem = sems[4]
    c = lax.axis_index("c")
    s = lax.axis_index("s")
    wid = s * NC + c
    b = wid // 8

    def start_idx(j, k, sl):
        pltpu.async_copy(idx_hbm.at[j, pl.ds(CHOFF[k], CHB[k])],
                         idx_v.at[sl, pl.ds(0, CHB[k])], isem[sl])

    def wait_idx(k, sl):
        pltpu.make_async_copy(idx_hbm.at[0, pl.ds(0, CHB[k])],
                              idx_v.at[sl, pl.ds(0, CHB[k])], isem[sl]).wait()

    def start_out(cb, ci, k, sl):
        pltpu.async_copy(out_v.at[sl, pl.ds(0, CHB[k])],
                         out_hbm.at[b, cb, pl.ds(CHOFF[k], CHB[k]), ci],
                         osem[sl])

    def wait_out(k, sl):
        pltpu.make_async_copy(out_v.at[sl, pl.ds(0, CHB[k])],
                              out_hbm.at[0, 0, pl.ds(0, CHB[k]), 0],
                              osem[sl]).wait()

    def gather_chunk(k, sl):
        @plsc.parallel_loop(0, CHB[k], unroll=4)
        def _vec8(t):
            for u in range(8):
                iv = idx_v[sl, t, pl.ds(u * 16, 16)]
                hi = lax.shift_right_logical(iv, 7)
                lo = lax.bitwise_and(iv, 127)
                out_v[sl, t, pl.ds(u * 16, 16)] = plsc.load_gather(
                    zcol_v, [hi, lo])

    def wait_z():
        pltpu.make_async_copy(zv_hbm.at[0, pl.ds(0, VB), 0], zcol_v,
                              zsem).wait()

    pltpu.async_copy(zv_hbm.at[wid, pl.ds(0, VB), 0], zcol_v, zsem)
    start_idx(0, 0, 0)

    NP = 8 * NU

    def body(p, carry):
        ci = p // NU
        j = lax.rem(p, NU)
        jn = lax.rem(p + 1, NU)
        cb = j * 8 + lax.rem(wid, 8)

        @pl.when(j == 0)
        def _():
            wait_z()

        for k in range(4):
            sl = k & 1
            if k < 3:
                start_idx(j, k + 1, sl ^ 1)
            wait_idx(k, sl)
            if k < 2:
                @pl.when(p > 0)
                def _():
                    wait_out(k + 2, sl)
            else:
                wait_out(k - 2, sl)
            gather_chunk(k, sl)
            start_out(cb, ci, k, sl)
            if k == 2:
                @pl.when(p < NP - 1)
                def _():
                    start_idx(jn, 0, 0)

        @pl.when((j == NU - 1) & (ci < 7))
        def _():
            pltpu.async_copy(zv_hbm.at[wid, pl.ds(0, VB), ci + 1], zcol_v,
                             zsem)

        return carry

    lax.fori_loop(0, NP, body, 0)
    wait_out(2, 0)
    wait_out(3, 1)


def kernel(z_prime, x_ancil, index):
    del x_ancil
    zv = z_prime.reshape(B, VB, 128, 8, 8).transpose(0, 3, 1, 4, 2)
    zv = zv.reshape(NW, VB, 8, 128)
    idx_t = jnp.transpose(index.astype(jnp.int32), (1, 0))
    idx_p = jnp.pad(idx_t, ((0, 0), (0, IPAD - N_VERTEX))).reshape(NU, IB, 128)
    out5 = _col_gather(zv, idx_p)
    out = out5.transpose(0, 1, 3, 2, 4).reshape(B, NU * D, IPAD)
    return out[:, :, :N_VERTEX].transpose(0, 2, 1)
